# Initial kernel scaffold; baseline (speedup 1.0000x reference)
import functools

import jax
import jax.numpy as jnp
from jax import lax
from jax.experimental import pallas as pl
from jax.experimental.pallas import tpu as pltpu
from jax.experimental.pallas import tpu_sc as plsc

N = 10000
D = 128
E = 320000
NC = 2
NS = 16
NW = NC * NS
EPW = E // NW
WIN = 80
NWIN = EPW // WIN
RPT = N // NS
ZR = 125

_mesh = plsc.VectorSubcoreMesh(core_axis_name="c", subcore_axis_name="s")


def _fill(ref, rows, cols, val):
    v = jnp.full((16,), val, jnp.float32)

    @pl.loop(0, rows)
    def _(i):
        @pl.loop(0, cols, step=16)
        def _(j):
            ref[i, pl.ds(j, 16)] = v


@functools.partial(
    pl.kernel,
    out_type=jax.ShapeDtypeStruct((NC, N, 16), jnp.float32),
    mesh=_mesh,
    scratch_types=[
        pltpu.VMEM((NWIN, WIN), jnp.int32),
        pltpu.VMEM((WIN, 16), jnp.float32),
        pltpu.VMEM((RPT, 16), jnp.float32),
        pltpu.VMEM_SHARED((N, 16), jnp.float32),
    ],
)
def _hist_kernel(col_hbm, out_hbm, idx_v, ones_v, zrows_v, acc_sh):
    cid = lax.axis_index("c")
    sid = lax.axis_index("s")
    wid = cid * NS + sid
    _fill(zrows_v, RPT, 16, 0.0)
    _fill(ones_v, WIN, 16, 1.0)
    pltpu.sync_copy(zrows_v, acc_sh.at[pl.ds(sid * RPT, RPT)])
    plsc.subcore_barrier()
    pltpu.sync_copy(col_hbm.at[pl.ds(wid * NWIN, NWIN)], idx_v)

    @pl.loop(0, NWIN)
    def _(j):
        pltpu.sync_copy(ones_v, acc_sh.at[idx_v.at[j]], add=True)

    plsc.subcore_barrier()
    pltpu.sync_copy(
        acc_sh.at[pl.ds(sid * RPT, RPT)],
        out_hbm.at[cid].at[pl.ds(sid * RPT, RPT)],
    )


@functools.partial(
    pl.kernel,
    out_type=jax.ShapeDtypeStruct((NC, N, D), jnp.float32),
    mesh=_mesh,
    scratch_types=[
        pltpu.VMEM((NWIN, WIN), jnp.int32),
        pltpu.VMEM((NWIN, WIN), jnp.int32),
        pltpu.VMEM((WIN, D), jnp.float32),
        pltpu.VMEM((ZR, D), jnp.float32),
        pltpu.VMEM_SHARED((N, D), jnp.float32),
    ],
)
def _agg_kernel(y_hbm, row_hbm, col_hbm, out_hbm, idxr_v, idxc_v, rows_v,
                zrows_v, acc_sh):
    cid = lax.axis_index("c")
    sid = lax.axis_index("s")
    wid = cid * NS + sid
    _fill(zrows_v, ZR, D, 0.0)

    @pl.loop(0, RPT, step=ZR)
    def _(r):
        pltpu.sync_copy(zrows_v, acc_sh.at[pl.ds(sid * RPT + r, ZR)])

    plsc.subcore_barrier()
    pltpu.sync_copy(row_hbm.at[pl.ds(wid * NWIN, NWIN)], idxr_v)
    pltpu.sync_copy(col_hbm.at[pl.ds(wid * NWIN, NWIN)], idxc_v)

    @pl.loop(0, NWIN)
    def _(j):
        pltpu.sync_copy(y_hbm.at[idxr_v.at[j]], rows_v)
        pltpu.sync_copy(rows_v, acc_sh.at[idxc_v.at[j]], add=True)

    plsc.subcore_barrier()

    @pl.loop(0, RPT, step=ZR)
    def _(r):
        pltpu.sync_copy(
            acc_sh.at[pl.ds(sid * RPT + r, ZR)],
            out_hbm.at[cid].at[pl.ds(sid * RPT + r, ZR)],
        )


def _dot(a, b):
    return lax.dot_general(a, b, (((1,), (0,)), ((), ())),
                           precision=lax.Precision.HIGHEST,
                           preferred_element_type=jnp.float32)


def _dis_from_hist(hist_ref):
    deg = hist_ref[0, :, 0:1] + hist_ref[1, :, 0:1] + 1.0
    return lax.rsqrt(deg)


def _mm_body(x_ref, w_ref, o_ref):
    o_ref[...] = _dot(x_ref[...], w_ref[...])


def _scale_body(hist_ref, xw_ref, o_ref):
    o_ref[...] = xw_ref[...] * _dis_from_hist(hist_ref)


def _mid_body(hist_ref, s_ref, y_ref, w_ref, b_ref, o_ref):
    dis = _dis_from_hist(hist_ref)
    h = jnp.maximum(dis * (s_ref[0] + s_ref[1] + y_ref[...]) + b_ref[...], 0.0)
    o_ref[...] = dis * _dot(h, w_ref[...])


def _final_body(hist_ref, s_ref, y_ref, b_ref, o_ref):
    dis = _dis_from_hist(hist_ref)
    o_ref[...] = dis * (s_ref[0] + s_ref[1] + y_ref[...]) + b_ref[...]


_nd_f32 = jax.ShapeDtypeStruct((N, D), jnp.float32)


def kernel(x, edge_index, W1, b1, W2, b2):
    row = edge_index[0].astype(jnp.int32).reshape(E // WIN, WIN)
    col = edge_index[1].astype(jnp.int32).reshape(E // WIN, WIN)
    b1r = b1.reshape(1, D)
    b2r = b2.reshape(1, D)

    hist = _hist_kernel(col)
    xw1 = pl.pallas_call(_mm_body, out_shape=_nd_f32)(x, W1)
    y1 = pl.pallas_call(_scale_body, out_shape=_nd_f32)(hist, xw1)
    s1 = _agg_kernel(y1, row, col)
    y2 = pl.pallas_call(_mid_body, out_shape=_nd_f32)(hist, s1, y1, W2, b1r)
    s2 = _agg_kernel(y2, row, col)
    out = pl.pallas_call(_final_body, out_shape=_nd_f32)(hist, s2, y2, b2r)
    return out


# speedup vs baseline: 18.7786x; 18.7786x over previous
import functools

import jax
import jax.numpy as jnp
from jax import lax
from jax.experimental import pallas as pl
from jax.experimental.pallas import tpu as pltpu
from jax.experimental.pallas import tpu_sc as plsc

N = 10000
D = 128
E = 320000
NC = 2
NS = 16
NW = NC * NS
EPW = E // NW
WIN = 80
NWIN = EPW // WIN
CH = 80
NCH = N // CH
CPS = -(-NCH // NS)

_mesh = plsc.VectorSubcoreMesh(core_axis_name="c", subcore_axis_name="s")


def _fill(ref, rows, cols, val):
    v = jnp.full((16,), val, jnp.float32)

    @pl.loop(0, rows)
    def _(i):
        @pl.loop(0, cols, step=16)
        def _(j):
            ref[i, pl.ds(j, 16)] = v


def _chunk_loop(sid, body):
    @pl.loop(0, CPS)
    def _(k):
        c = k * NS + sid

        @pl.when(c < NCH)
        def _():
            body(c * CH)


@functools.partial(
    pl.kernel,
    out_type=jax.ShapeDtypeStruct((NC, N, D), jnp.float32),
    mesh=_mesh,
    scratch_types=[
        pltpu.VMEM((NWIN, WIN), jnp.int32),
        pltpu.VMEM((WIN, D), jnp.float32),
        pltpu.VMEM((CH, D), jnp.float32),
        pltpu.VMEM_SHARED((N, D), jnp.float32),
    ],
)
def _hist_kernel(col_hbm, out_hbm, idx_v, ones_v, zrows_v, acc_sh):
    cid = lax.axis_index("c")
    sid = lax.axis_index("s")
    wid = cid * NS + sid
    _fill(zrows_v, CH, D, 0.0)
    _fill(ones_v, WIN, D, 1.0)
    _chunk_loop(sid, lambda r: pltpu.sync_copy(zrows_v, acc_sh.at[pl.ds(r, CH)]))
    plsc.subcore_barrier()
    pltpu.sync_copy(col_hbm.at[wid], idx_v)

    @pl.loop(0, NWIN)
    def _(j):
        pltpu.sync_copy(ones_v, acc_sh.at[idx_v.at[j]], add=True)

    plsc.subcore_barrier()
    _chunk_loop(sid, lambda r: pltpu.sync_copy(
        acc_sh.at[pl.ds(r, CH)], out_hbm.at[cid].at[pl.ds(r, CH)]))


@functools.partial(
    pl.kernel,
    out_type=jax.ShapeDtypeStruct((NC, N, D), jnp.float32),
    mesh=_mesh,
    scratch_types=[
        pltpu.VMEM((NWIN, WIN), jnp.int32),
        pltpu.VMEM((NWIN, WIN), jnp.int32),
        pltpu.VMEM((WIN, D), jnp.float32),
        pltpu.VMEM_SHARED((N, D), jnp.float32),
    ],
)
def _agg_kernel(y_hbm, row_hbm, col_hbm, out_hbm, idxr_v, idxc_v, rows_v,
                acc_sh):
    cid = lax.axis_index("c")
    sid = lax.axis_index("s")
    wid = cid * NS + sid
    _fill(rows_v, CH, D, 0.0)
    _chunk_loop(sid, lambda r: pltpu.sync_copy(rows_v, acc_sh.at[pl.ds(r, CH)]))
    plsc.subcore_barrier()
    pltpu.sync_copy(row_hbm.at[wid], idxr_v)
    pltpu.sync_copy(col_hbm.at[wid], idxc_v)

    @pl.loop(0, NWIN)
    def _(j):
        pltpu.sync_copy(y_hbm.at[idxr_v.at[j]], rows_v)
        pltpu.sync_copy(rows_v, acc_sh.at[idxc_v.at[j]], add=True)

    plsc.subcore_barrier()
    _chunk_loop(sid, lambda r: pltpu.sync_copy(
        acc_sh.at[pl.ds(r, CH)], out_hbm.at[cid].at[pl.ds(r, CH)]))


def _dot(a, b):
    return lax.dot_general(a, b, (((1,), (0,)), ((), ())),
                           precision=lax.Precision.HIGHEST,
                           preferred_element_type=jnp.float32)


def _dis_from_hist(hist_ref):
    deg = hist_ref[0, :, 0:1] + hist_ref[1, :, 0:1] + 1.0
    return lax.rsqrt(deg)


def _mm_body(x_ref, w_ref, o_ref):
    o_ref[...] = _dot(x_ref[...], w_ref[...])


def _scale_body(hist_ref, xw_ref, o_ref):
    o_ref[...] = xw_ref[...] * _dis_from_hist(hist_ref)


def _mid_body(hist_ref, s_ref, y_ref, w_ref, b_ref, o_ref):
    dis = _dis_from_hist(hist_ref)
    h = jnp.maximum(dis * (s_ref[0] + s_ref[1] + y_ref[...]) + b_ref[...], 0.0)
    o_ref[...] = dis * _dot(h, w_ref[...])


def _final_body(hist_ref, s_ref, y_ref, b_ref, o_ref):
    dis = _dis_from_hist(hist_ref)
    o_ref[...] = dis * (s_ref[0] + s_ref[1] + y_ref[...]) + b_ref[...]


_nd_f32 = jax.ShapeDtypeStruct((N, D), jnp.float32)


def kernel(x, edge_index, W1, b1, W2, b2):
    row = edge_index[0].astype(jnp.int32).reshape(NW, NWIN, WIN)
    col = edge_index[1].astype(jnp.int32).reshape(NW, NWIN, WIN)
    b1r = b1.reshape(1, D)
    b2r = b2.reshape(1, D)

    hist = _hist_kernel(col)
    xw1 = pl.pallas_call(_mm_body, out_shape=_nd_f32)(x, W1)
    y1 = pl.pallas_call(_scale_body, out_shape=_nd_f32)(hist, xw1)
    s1 = _agg_kernel(y1, row, col)
    y2 = pl.pallas_call(_mid_body, out_shape=_nd_f32)(hist, s1, y1, W2, b1r)
    s2 = _agg_kernel(y2, row, col)
    out = pl.pallas_call(_final_body, out_shape=_nd_f32)(hist, s2, y2, b2r)
    return out
